# baseline (speedup 1.0000x reference)
import jax
import jax.numpy as jnp
from jax.experimental import pallas as pl
from jax.experimental.pallas import tpu as pltpu


def _dot(a, b):
    return jax.lax.dot_general(
        a, b, (((1,), (0,)), ((), ())),
        precision=jax.lax.Precision.DEFAULT,
        preferred_element_type=jnp.float32)


def _head_kernel(ea_ref, p_ref, out_ref):
    p = p_ref[...]
    w1, b1 = p[:, 0:4], p[:, 4:5]
    w2, b2 = p[:, 5:45], p[:, 45:46]
    w3, b3 = p[:, 46:86], p[:, 86:87]
    w4t, b4 = p[:, 87:88], p[0:1, 88:89]
    h = jnp.maximum(_dot(w1, ea_ref[...].astype(jnp.float32)) + b1, 0.0)
    h = jnp.maximum(_dot(w2, h) + b2, 0.0)
    h = jnp.maximum(_dot(w3, h) + b3, 0.0)
    o = jax.lax.dot_general(
        w4t, h, (((0,), (0,)), ((), ())),
        precision=jax.lax.Precision.DEFAULT,
        preferred_element_type=jnp.float32)
    out_ref[...] = jax.nn.sigmoid(o + b4)


def kernel(x, edge_index, edge_attr, params_rel, params_obj, params_w):
    E, DE = edge_attr.shape
    (W1, b1), (W2, b2), (W3, b3), (W4, b4) = params_w
    H = W1.shape[0]
    DO = W4.shape[0]

    eaT = edge_attr.T.astype(jnp.bfloat16)
    packed = jnp.concatenate(
        [W1, b1[:, None], W2, b2[:, None], W3, b3[:, None], W4.T,
         jnp.full((H, 1), b4[0], jnp.float32)], axis=1)

    lanes = 32000
    grid = (pl.cdiv(E, lanes),)

    out = pl.pallas_call(
        _head_kernel,
        grid=grid,
        in_specs=[
            pl.BlockSpec((DE, lanes), lambda i: (0, i)),
            pl.BlockSpec(packed.shape, lambda i: (0, 0)),
        ],
        out_specs=pl.BlockSpec((DO, lanes), lambda i: (0, i)),
        out_shape=jax.ShapeDtypeStruct((DO, E), jnp.float32),
        compiler_params=pltpu.CompilerParams(
            dimension_semantics=("parallel",)),
    )(eaT, packed)
    return out.reshape(E, DO)


# speedup vs baseline: 1.1432x; 1.1432x over previous
import jax
import jax.numpy as jnp
from jax.experimental import pallas as pl
from jax.experimental.pallas import tpu as pltpu


def _dot(a, b):
    return jax.lax.dot_general(
        a, b, (((1,), (0,)), ((), ())),
        precision=jax.lax.Precision.DEFAULT,
        preferred_element_type=jnp.float32)


def _head_kernel(ea_ref, p_ref, out_ref):
    p = p_ref[...]
    w1, b1 = p[:, 0:4], p[:, 4:5]
    w2, b2 = p[:, 5:45], p[:, 45:46]
    w3, b3 = p[:, 46:86], p[:, 86:87]
    w4t, b4 = p[:, 87:88], p[0:1, 88:89]
    h = jnp.maximum(_dot(w1, ea_ref[...]) + b1, 0.0)
    h = jnp.maximum(_dot(w2, h) + b2, 0.0)
    h = jnp.maximum(_dot(w3, h) + b3, 0.0)
    o = jax.lax.dot_general(
        w4t, h, (((0,), (0,)), ((), ())),
        precision=jax.lax.Precision.DEFAULT,
        preferred_element_type=jnp.float32)
    out_ref[...] = jax.nn.sigmoid(o + b4)


def kernel(x, edge_index, edge_attr, params_rel, params_obj, params_w):
    E, DE = edge_attr.shape
    (W1, b1), (W2, b2), (W3, b3), (W4, b4) = params_w
    H = W1.shape[0]
    DO = W4.shape[0]

    eaT = edge_attr.T
    packed = jnp.concatenate(
        [W1, b1[:, None], W2, b2[:, None], W3, b3[:, None], W4.T,
         jnp.full((H, 1), b4[0], jnp.float32)], axis=1)

    lanes = 32000
    grid = (pl.cdiv(E, lanes),)

    out = pl.pallas_call(
        _head_kernel,
        grid=grid,
        in_specs=[
            pl.BlockSpec((DE, lanes), lambda i: (0, i)),
            pl.BlockSpec(packed.shape, lambda i: (0, 0)),
        ],
        out_specs=pl.BlockSpec((DO, lanes), lambda i: (0, i)),
        out_shape=jax.ShapeDtypeStruct((DO, E), jnp.float32),
        compiler_params=pltpu.CompilerParams(
            dimension_semantics=("parallel",)),
    )(eaT, packed)
    return out.reshape(E, DO)


# speedup vs baseline: 1.1448x; 1.0014x over previous
import jax
import jax.numpy as jnp
from jax.experimental import pallas as pl
from jax.experimental.pallas import tpu as pltpu


def _dot(a, b):
    return jax.lax.dot_general(
        a, b, (((1,), (0,)), ((), ())),
        precision=jax.lax.Precision.DEFAULT,
        preferred_element_type=jnp.float32)


def _make_head_kernel(de, h1, h2, h3):
    o1 = de + 1
    o2 = o1 + h1 + 1
    o3 = o2 + h2 + 1

    def _head_kernel(ea_ref, p_ref, out_ref):
        p = p_ref[...]
        w1, b1 = p[:, 0:de], p[:, de:de + 1]
        w2, b2 = p[:, o1:o1 + h1], p[:, o1 + h1:o1 + h1 + 1]
        w3, b3 = p[:, o2:o2 + h2], p[:, o2 + h2:o2 + h2 + 1]
        w4t, b4 = p[:, o3:o3 + 1], p[0:1, o3 + 1:o3 + 2]
        h = jnp.maximum(_dot(w1, ea_ref[...]) + b1, 0.0)
        h = jnp.maximum(_dot(w2, h) + b2, 0.0)
        h = jnp.maximum(_dot(w3, h) + b3, 0.0)
        o = jax.lax.dot_general(
            w4t, h, (((0,), (0,)), ((), ())),
            precision=jax.lax.Precision.DEFAULT,
            preferred_element_type=jnp.float32)
        out_ref[...] = jax.nn.sigmoid(o + b4)

    return _head_kernel


def kernel(x, edge_index, edge_attr, params_rel, params_obj, params_w):
    E, DE = edge_attr.shape
    (W1, b1), (W2, b2), (W3, b3), (W4, b4) = params_w
    H = W1.shape[0]
    DO = W4.shape[0]

    eaT = edge_attr.T
    packed = jnp.concatenate(
        [W1, b1[:, None], W2, b2[:, None], W3, b3[:, None], W4.T,
         jnp.full((H, 1), b4[0], jnp.float32)], axis=1)

    lanes = 32000
    grid = (pl.cdiv(E, lanes),)

    out = pl.pallas_call(
        _make_head_kernel(DE, W2.shape[1], W3.shape[1], W4.shape[1]),
        grid=grid,
        in_specs=[
            pl.BlockSpec((DE, lanes), lambda i: (0, i)),
            pl.BlockSpec(packed.shape, lambda i: (0, 0)),
        ],
        out_specs=pl.BlockSpec((DO, lanes), lambda i: (0, i)),
        out_shape=jax.ShapeDtypeStruct((DO, E), jnp.float32),
        compiler_params=pltpu.CompilerParams(
            dimension_semantics=("parallel",)),
    )(eaT, packed)
    return out.reshape(E, DO)
